# baseline (speedup 1.0000x reference)
import functools

import jax
import jax.numpy as jnp
from jax import lax
from jax.experimental import pallas as pl
from jax.experimental.pallas import tpu as pltpu
from jax.experimental.pallas import tpu_sc as plsc

N = 1966080
BETA = 1.0 / 9.0

NC = 2
NS = 16
L = 16
NW = NC * NS
SC_CHUNK = 3072
SC_CHUNKS = 12
SC_PER_W = SC_CHUNK * SC_CHUNKS
N_SC = SC_PER_W * NW
N_TC = N - N_SC

LANES = 128
ROWS = N // LANES
BR = 512
TC_LOC_BLOCKS = ROWS // BR
BCE_ROW0 = N_SC // LANES
TC_BCE_BLOCKS = (ROWS - BCE_ROW0) // BR

assert SC_CHUNKS % 2 == 0
assert N_SC % (LANES * BR) == 0 and N_TC % (LANES * BR) == 0

_mesh = plsc.VectorSubcoreMesh(
    core_axis_name="c", subcore_axis_name="s", num_cores=NC, num_subcores=NS
)


def _vf(c):
    return jnp.full((L,), c, jnp.float32)


def _vi(c):
    return jnp.full((L,), c, jnp.int32)


@functools.partial(
    pl.kernel,
    out_type=jax.ShapeDtypeStruct((NW * 32,), jnp.float32),
    mesh=_mesh,
    scratch_types=[
        pltpu.VMEM((2, SC_CHUNK), jnp.float32),
        pltpu.VMEM((2, SC_CHUNK), jnp.int32),
        pltpu.VMEM((32,), jnp.float32),
        pltpu.SemaphoreType.DMA,
        pltpu.SemaphoreType.DMA,
    ],
)
def _rpn_sc_bce(obj_hbm, lab_hbm, out_hbm, obj_v, lab_v, part_v, sem0, sem1):
    wid = lax.axis_index("c") * NS + lax.axis_index("s")
    base = wid * SC_PER_W
    sems = (sem0, sem1)

    def _copies(g, slot):
        sem = sems[slot]
        return (
            pltpu.make_async_copy(
                obj_hbm.at[pl.ds(base + g * SC_CHUNK, SC_CHUNK)], obj_v.at[slot], sem),
            pltpu.make_async_copy(
                lab_hbm.at[pl.ds(base + g * SC_CHUNK, SC_CHUNK)], lab_v.at[slot], sem),
        )

    def _start(g, slot):
        for c in _copies(g, slot):
            c.start()

    def _wait(g, slot):
        for c in _copies(g, slot):
            c.wait()

    def _bce_group(i, obj_ref, lab_ref, accs):
        acc_cls, acc_val = accs
        x = obj_ref[pl.ds(i * L, L)]
        lab = lab_ref[pl.ds(i * L, L)]
        ones, zeros = _vf(1.0), _vf(0.0)
        posf = jnp.where(lab == _vi(1), ones, zeros)
        validf = jnp.where(lab != _vi(2), ones, zeros)
        t = jnp.exp(zeros - jnp.abs(x))
        z = t / (_vf(2.0) + t)
        w = z * z
        l1p = (_vf(2.0) * z) * (
            ones + w * (_vf(1.0 / 3.0)
                        + w * (_vf(1.0 / 5.0)
                               + w * (_vf(1.0 / 7.0) + w * _vf(1.0 / 9.0))))
        )
        ce = jnp.maximum(x, zeros) - x * posf + l1p
        return acc_cls + ce * validf, acc_val + validf

    def _compute(slot, accs):
        return lax.fori_loop(
            0, SC_CHUNK // L,
            lambda i, cv: _bce_group(i, obj_v.at[slot], lab_v.at[slot], cv),
            accs,
        )

    zero = jnp.zeros((L,), jnp.float32)
    _start(0, 0)

    def outer(o, accs):
        g0 = 2 * o
        _start(g0 + 1, 1)
        _wait(g0, 0)
        accs = _compute(0, accs)

        @pl.when(o < SC_CHUNKS // 2 - 1)
        def _():
            _start(g0 + 2, 0)

        _wait(g0 + 1, 1)
        return _compute(1, accs)

    acc_cls, acc_val = lax.fori_loop(0, SC_CHUNKS // 2, outer, (zero, zero))

    part_v[pl.ds(0, L)] = acc_cls
    part_v[pl.ds(16, L)] = acc_val
    pltpu.sync_copy(part_v, out_hbm.at[pl.ds(wid * 32, 32)])


def _tc_loc_body(lab_ref, pred_ref, gt_ref, loc_ref):
    i = pl.program_id(0)
    d = pred_ref[...] - gt_ref[...]
    a = jnp.abs(d)
    sl1 = jnp.where(a < BETA, (0.5 / BETA) * d * d, a - 0.5 * BETA)
    comp = lax.broadcasted_iota(jnp.int32, (4 * LANES, LANES), 0)
    anch = lax.broadcasted_iota(jnp.int32, (4 * LANES, LANES), 1)
    fmat = (comp // 4 == anch).astype(jnp.float32)
    s = jax.lax.dot(sl1, fmat, preferred_element_type=jnp.float32)
    labf = (lab_ref[...] == 1).astype(jnp.float32)
    part = jnp.sum(s * labf)

    @pl.when(i == 0)
    def _():
        loc_ref[0, 0] = 0.0

    loc_ref[0, 0] += part


def _tc_bce_body(obj_ref, lab_ref, cls_ref, val_ref):
    i = pl.program_id(0)
    x = obj_ref[...]
    lab = lab_ref[...]
    posf = (lab == 1).astype(jnp.float32)
    validf = (lab != 2).astype(jnp.float32)
    ce = jnp.maximum(x, 0.0) - x * posf + jnp.log(1.0 + jnp.exp(-jnp.abs(x)))
    pc = jnp.sum(ce * validf)
    pv = jnp.sum(validf)

    @pl.when(i == 0)
    def _():
        cls_ref[0, 0] = 0.0
        val_ref[0, 0] = 0.0

    cls_ref[0, 0] += pc
    val_ref[0, 0] += pv


def kernel(pred_objectness_logits, pred_anchor_deltas, gt_anchor_deltas, gt_labels):
    obj2 = pred_objectness_logits.reshape(ROWS, LANES)
    lab2 = gt_labels.reshape(ROWS, LANES)
    pred2 = pred_anchor_deltas.reshape(ROWS, 4 * LANES)
    gt2 = gt_anchor_deltas.reshape(ROWS, 4 * LANES)

    sc_parts = _rpn_sc_bce(pred_objectness_logits, gt_labels)

    loc = pl.pallas_call(
        _tc_loc_body,
        grid=(TC_LOC_BLOCKS,),
        in_specs=[
            pl.BlockSpec((BR, LANES), lambda i: (i, 0)),
            pl.BlockSpec((BR, 4 * LANES), lambda i: (i, 0)),
            pl.BlockSpec((BR, 4 * LANES), lambda i: (i, 0)),
        ],
        out_specs=pl.BlockSpec(memory_space=pltpu.SMEM),
        out_shape=jax.ShapeDtypeStruct((1, 1), jnp.float32),
    )(lab2, pred2, gt2)

    cls_tc, val_tc = pl.pallas_call(
        _tc_bce_body,
        grid=(TC_BCE_BLOCKS,),
        in_specs=[
            pl.BlockSpec((BR, LANES), lambda i: (BCE_ROW0 // BR + i, 0)),
            pl.BlockSpec((BR, LANES), lambda i: (BCE_ROW0 // BR + i, 0)),
        ],
        out_specs=[
            pl.BlockSpec(memory_space=pltpu.SMEM),
            pl.BlockSpec(memory_space=pltpu.SMEM),
        ],
        out_shape=[
            jax.ShapeDtypeStruct((1, 1), jnp.float32),
            jax.ShapeDtypeStruct((1, 1), jnp.float32),
        ],
    )(obj2, lab2)

    p = sc_parts.reshape(NW, 2, L)
    loss_cls = jnp.sum(p[:, 0, :]) + cls_tc[0, 0]
    valid = jnp.sum(p[:, 1, :]) + val_tc[0, 0]
    total = (loss_cls + loc[0, 0]) / jnp.maximum(valid, 1.0)
    return total
